# baseline (speedup 1.0000x reference)
import functools

import jax
import jax.numpy as jnp
from jax import lax
from jax.experimental import pallas as pl
from jax.experimental.pallas import tpu as pltpu
from jax.experimental.pallas import tpu_sc as plsc

DIM = 64
LANES = 16
NJ = DIM // LANES
NC = 2
NS = 16
NW = NC * NS
CHUNK = 128
UNROLL = 4


def _rsqrt16(x):
    i = plsc.bitcast(x, jnp.int32)
    i = jnp.int32(0x5F3759DF) - (i >> 1)
    y = plsc.bitcast(i, jnp.float32)
    half = x * jnp.float32(0.5)
    for _ in range(3):
        y = y * (jnp.float32(1.5) - half * y * y)
    return y


def _ln_chunk(e_ref, t_ref, o_ref, gvecs, bvecs):
    iota = lax.iota(jnp.int32, 16)
    dtr = [(16 * j + iota) >> 3 for j in range(NJ)]
    din = [(16 * j + iota) & 7 for j in range(NJ)]

    def body(it, carry):
        base = it * UNROLL
        for u in range(UNROLL):
            r = base + u
            h = [e_ref[r, pl.ds(16 * j, 16)] + t_ref[r, pl.ds(16 * j, 16)]
                 for j in range(NJ)]
            s = (h[0] + h[1]) + (h[2] + h[3])
            ss = (h[0] * h[0] + h[1] * h[1]) + (h[2] * h[2] + h[3] * h[3])
            tot = jnp.sum(s)
            sstot = jnp.sum(ss)
            mu = tot * jnp.float32(1.0 / DIM)
            var = sstot * jnp.float32(1.0 / DIM) - mu * mu
            xv = jnp.full((16,), var + jnp.float32(1e-5), jnp.float32)
            rstd = _rsqrt16(xv)
            muv = jnp.full((16,), mu, jnp.float32)
            rv = jnp.full((16,), r, jnp.int32)
            for j in range(NJ):
                val = (h[j] - muv) * rstd * gvecs[j] + bvecs[j]
                plsc.store_scatter(o_ref, [dtr[j], din[j], rv], val)
        return carry

    lax.fori_loop(0, CHUNK // UNROLL, body, 0, unroll=False)


def _sc_body(idx_hbm, ts_hbm, table_hbm, ttable_hbm, gamma_hbm, beta_hbm,
             out_hbm, idx_v, ts_v, e0, e1, t0, t1, o0, o1, o2, o3, gam_v, bet_v,
             se0, se1, st0, st1, so0, so1, so2, so3, nunit, nbk):
    wid = lax.axis_index("s") * NC + lax.axis_index("c")
    ubase = wid * nunit

    pltpu.sync_copy(idx_hbm.at[wid], idx_v)
    pltpu.sync_copy(ts_hbm.at[wid], ts_v)
    pltpu.sync_copy(gamma_hbm, gam_v)
    pltpu.sync_copy(beta_hbm, bet_v)
    gvecs = [gam_v[pl.ds(16 * j, 16)] for j in range(NJ)]
    bvecs = [bet_v[pl.ds(16 * j, 16)] for j in range(NJ)]

    ebufs, tbufs, obufs = (e0, e1), (t0, t1), (o0, o1, o2, o3)
    esems, tsems, osems = (se0, se1), (st0, st1), (so0, so1, so2, so3)

    def issue_gather(i, p):
        pltpu.async_copy(table_hbm.at[idx_v.at[i]], ebufs[p], esems[p])
        pltpu.async_copy(ttable_hbm.at[ts_v.at[i]], tbufs[p], tsems[p])

    def wait_gather(i, p):
        pltpu.make_async_copy(table_hbm.at[idx_v.at[i]], ebufs[p], esems[p]).wait()
        pltpu.make_async_copy(ttable_hbm.at[ts_v.at[i]], tbufs[p], tsems[p]).wait()

    def issue_out(i, q):
        ug = ubase + i
        l0, kb = ug // nbk, ug % nbk
        for d in range(DIM // 8):
            pltpu.async_copy(obufs[q].at[d, :, pl.ds(0, CHUNK)],
                             out_hbm.at[l0, d, kb], osems[q])

    def wait_out(i, q):
        ug = ubase + i
        l0, kb = ug // nbk, ug % nbk
        for d in range(DIM // 8):
            pltpu.make_async_copy(obufs[q].at[d, :, pl.ds(0, CHUNK)],
                                  out_hbm.at[l0, d, kb], osems[q]).wait()

    def step(i, p, q, issue_next, wait_prev_scatter):
        wait_gather(i, p)
        if issue_next:
            issue_gather(i + 1, 1 - p)
        if wait_prev_scatter:
            wait_out(i - 4, q)
        _ln_chunk(ebufs[p], tbufs[p], obufs[q], gvecs, bvecs)
        issue_out(i, q)

    issue_gather(0, 0)
    step(0, 0, 0, True, False)
    step(1, 1, 1, True, False)
    step(2, 0, 2, True, False)
    step(3, 1, 3, True, False)

    def loop_body(g, carry):
        i = 4 * g
        step(i, 0, 0, True, True)
        step(i + 1, 1, 1, True, True)
        step(i + 2, 0, 2, True, True)
        step(i + 3, 1, 3, True, True)
        return carry

    lax.fori_loop(1, (nunit - 4) // 4, loop_body, 0, unroll=False)

    step(nunit - 4, 0, 0, True, True)
    step(nunit - 3, 1, 1, True, True)
    step(nunit - 2, 0, 2, True, True)
    step(nunit - 1, 1, 3, False, True)
    wait_out(nunit - 4, 0)
    wait_out(nunit - 3, 1)
    wait_out(nunit - 2, 2)
    wait_out(nunit - 1, 3)


@functools.partial(jax.jit, static_argnames=("nl", "nb", "nunit"))
def _run(idx3, ts3, table, time_table, gamma, beta, nl, nb, nunit):
    mesh = plsc.VectorSubcoreMesh(core_axis_name="c", subcore_axis_name="s",
                                  num_cores=NC, num_subcores=NS)
    body = functools.partial(_sc_body, nunit=nunit, nbk=nb // CHUNK)
    f = pl.kernel(
        body,
        out_type=jax.ShapeDtypeStruct((nl, DIM // 8, nb // CHUNK, 8, CHUNK),
                                      jnp.float32),
        mesh=mesh,
        compiler_params=pltpu.CompilerParams(
            needs_layout_passes=False, use_tc_tiling_on_sc=False),
        scratch_types=[
            pltpu.VMEM((nunit, CHUNK), jnp.int32),
            pltpu.VMEM((nunit, CHUNK), jnp.int32),
            pltpu.VMEM((CHUNK, DIM), jnp.float32),
            pltpu.VMEM((CHUNK, DIM), jnp.float32),
            pltpu.VMEM((CHUNK, DIM), jnp.float32),
            pltpu.VMEM((CHUNK, DIM), jnp.float32),
            pltpu.VMEM((DIM // 8, 8, CHUNK + 1), jnp.float32),
            pltpu.VMEM((DIM // 8, 8, CHUNK + 1), jnp.float32),
            pltpu.VMEM((DIM // 8, 8, CHUNK + 1), jnp.float32),
            pltpu.VMEM((DIM // 8, 8, CHUNK + 1), jnp.float32),
            pltpu.VMEM((DIM,), jnp.float32),
            pltpu.VMEM((DIM,), jnp.float32),
            pltpu.SemaphoreType.DMA,
            pltpu.SemaphoreType.DMA,
            pltpu.SemaphoreType.DMA,
            pltpu.SemaphoreType.DMA,
            pltpu.SemaphoreType.DMA,
            pltpu.SemaphoreType.DMA,
            pltpu.SemaphoreType.DMA,
            pltpu.SemaphoreType.DMA,
        ],
    )
    return f(idx3, ts3, table, time_table, gamma, beta)


def kernel(indices, time_steps, table, time_table, gamma, beta):
    b, l = indices.shape
    n = b * l
    assert b % CHUNK == 0
    nunit = (l * b // CHUNK) // NW
    assert nunit * NW * CHUNK == n and nunit % 2 == 0
    idx3 = jnp.transpose(indices).astype(jnp.int32).reshape(NW, nunit, CHUNK)
    ts3 = jnp.transpose(time_steps).astype(jnp.int32).reshape(NW, nunit, CHUNK)
    out5 = _run(idx3, ts3, table.astype(jnp.float32),
                time_table.astype(jnp.float32), gamma.astype(jnp.float32),
                beta.astype(jnp.float32), l, b, nunit)
    out = jnp.transpose(out5, (2, 4, 0, 1, 3)).reshape(b, l, DIM)
    return out


# speedup vs baseline: 1.0119x; 1.0119x over previous
import functools

import jax
import jax.numpy as jnp
from jax import lax
from jax.experimental import pallas as pl
from jax.experimental.pallas import tpu as pltpu
from jax.experimental.pallas import tpu_sc as plsc

DIM = 64
LANES = 16
NC = 2
NS = 16
NW = NC * NS
CHUNK = 128
UC = 4
UNIT = UC * CHUNK


def _rsqrt16(x):
    i = plsc.bitcast(x, jnp.int32)
    i = jnp.int32(0x5F3759DF) - (i >> 1)
    y = plsc.bitcast(i, jnp.float32)
    half = x * jnp.float32(0.5)
    for _ in range(3):
        y = y * (jnp.float32(1.5) - half * y * y)
    return y


def _ln_chunk(e_ref, t_ref, o_ref, c, grot, brot, hbuf):
    iota = lax.iota(jnp.int32, 16)
    cv = jnp.full((16,), c, jnp.int32)

    def bg_body(bg, carry):
        bvec = bg * 16 + iota

        def p1(d0, acc):
            s, ss = acc
            dvec = (d0 + iota) & (DIM - 1)
            h = (plsc.load_gather(e_ref, [bvec, dvec])
                 + plsc.load_gather(t_ref, [bvec, dvec]))
            hbuf[d0] = h
            return s + h, ss + h * h

        zero = jnp.zeros((16,), jnp.float32)
        s, ss = lax.fori_loop(0, DIM, p1, (zero, zero), unroll=4)
        mu = s * jnp.float32(1.0 / DIM)
        var = ss * jnp.float32(1.0 / DIM) - mu * mu
        rstd = _rsqrt16(var + jnp.float32(1e-5))

        def p2(d0, carry2):
            dvec = (d0 + iota) & (DIM - 1)
            val = (hbuf[d0] - mu) * rstd * grot[d0] + brot[d0]
            plsc.store_scatter(o_ref, [dvec >> 3, cv, dvec & 7, bvec], val)
            return carry2

        lax.fori_loop(0, DIM, p2, 0, unroll=4)
        return carry

    lax.fori_loop(0, CHUNK // 16, bg_body, 0, unroll=False)


def _sc_body(idx_hbm, ts_hbm, table_hbm, ttable_hbm, gamma_hbm, beta_hbm,
             out_hbm, i0, i1, u0, u1, e0, e1, t0, t1, o0, o1,
             gam_v, bet_v, grot, brot, hbuf,
             si0, si1, su0, su1, se0, se1, st0, st1, so0, so1, nunit, nsb):
    wid = lax.axis_index("s") * NC + lax.axis_index("c")
    ubase = wid * nunit
    iota = lax.iota(jnp.int32, 16)

    pltpu.sync_copy(gamma_hbm, gam_v)
    pltpu.sync_copy(beta_hbm, bet_v)

    def rot_body(d0, carry):
        dvec = (d0 + iota) & (DIM - 1)
        grot[d0] = plsc.load_gather(gam_v, [dvec])
        brot[d0] = plsc.load_gather(bet_v, [dvec])
        return carry

    lax.fori_loop(0, DIM, rot_body, 0, unroll=False)

    ibufs, ubufs = (i0, i1), (u0, u1)
    ebufs, tbufs, obufs = (e0, e1), (t0, t1), (o0, o1)
    isems, usems = (si0, si1), (su0, su1)
    esems, tsems, osems = (se0, se1), (st0, st1), (so0, so1)

    def issue_idx(u, pu):
        pltpu.async_copy(idx_hbm.at[wid, u], ibufs[pu], isems[pu])
        pltpu.async_copy(ts_hbm.at[wid, u], ubufs[pu], usems[pu])

    def wait_idx(u, pu):
        pltpu.make_async_copy(idx_hbm.at[wid, u], ibufs[pu], isems[pu]).wait()
        pltpu.make_async_copy(ts_hbm.at[wid, u], ubufs[pu], usems[pu]).wait()

    def issue_gather(pu, c, p):
        sl = pl.ds(c * CHUNK, CHUNK)
        pltpu.async_copy(table_hbm.at[ibufs[pu].at[sl]], ebufs[p], esems[p])
        pltpu.async_copy(ttable_hbm.at[ubufs[pu].at[sl]], tbufs[p], tsems[p])

    def wait_gather(pu, c, p):
        sl = pl.ds(c * CHUNK, CHUNK)
        pltpu.make_async_copy(table_hbm.at[ibufs[pu].at[sl]], ebufs[p], esems[p]).wait()
        pltpu.make_async_copy(ttable_hbm.at[ubufs[pu].at[sl]], tbufs[p], tsems[p]).wait()

    def flush_out(u, pu):
        ug = ubase + u
        l0, sb = ug // nsb, ug % nsb
        for d in range(DIM // 8):
            pltpu.async_copy(obufs[pu].at[d],
                             out_hbm.at[l0, d, pl.ds(sb * UC, UC)], osems[pu])

    def wait_out(u, pu):
        ug = ubase + u
        l0, sb = ug // nsb, ug % nsb
        for d in range(DIM // 8):
            pltpu.make_async_copy(obufs[pu].at[d],
                                  out_hbm.at[l0, d, pl.ds(sb * UC, UC)],
                                  osems[pu]).wait()

    def chunk_step(u, pu, c, p, issue_next, wait_prev_flush, prefetch2=True):
        wait_gather(pu, c, p)
        if issue_next:
            if c == UC - 1:
                wait_idx(u + 1, 1 - pu)
                issue_gather(1 - pu, 0, 1 - p)
                if prefetch2:
                    issue_idx(u + 2, pu)
            else:
                issue_gather(pu, c + 1, 1 - p)
        if wait_prev_flush and c == 0:
            wait_out(u - 2, pu)
        _ln_chunk(ebufs[p], tbufs[p], obufs[pu], c, grot, brot, hbuf)
        if c == UC - 1:
            flush_out(u, pu)

    def unit_steps(u, pu, p0, issue_next, wait_prev_flush, issue_idx2):
        for c in range(UC):
            last = c == UC - 1
            chunk_step(u, pu, c, (p0 + c) % 2,
                       issue_next or not last, wait_prev_flush,
                       prefetch2=issue_idx2)

    issue_idx(0, 0)
    wait_idx(0, 0)
    issue_gather(0, 0, 0)
    issue_idx(1, 1)
    unit_steps(0, 0, 0, True, False, True)
    unit_steps(1, 1, 0, True, False, True)

    def loop_body(g, carry):
        u = 2 * g
        unit_steps(u, 0, 0, True, True, True)
        unit_steps(u + 1, 1, 0, True, True, True)
        return carry

    lax.fori_loop(1, (nunit - 2) // 2, loop_body, 0, unroll=False)

    unit_steps(nunit - 2, 0, 0, True, True, False)
    for c in range(UC):
        chunk_step(nunit - 1, 1, c, c % 2, c != UC - 1, True)
    wait_out(nunit - 2, 0)
    wait_out(nunit - 1, 1)


@functools.partial(jax.jit, static_argnames=("nl", "nb", "nunit"))
def _run(idx3, ts3, table, time_table, gamma, beta, nl, nb, nunit):
    mesh = plsc.VectorSubcoreMesh(core_axis_name="c", subcore_axis_name="s",
                                  num_cores=NC, num_subcores=NS)
    body = functools.partial(_sc_body, nunit=nunit, nsb=nb // UNIT)
    f = pl.kernel(
        body,
        out_type=jax.ShapeDtypeStruct((nl, DIM // 8, nb // CHUNK, 8, CHUNK),
                                      jnp.float32),
        mesh=mesh,
        compiler_params=pltpu.CompilerParams(
            needs_layout_passes=False, use_tc_tiling_on_sc=False),
        scratch_types=[
            pltpu.VMEM((UNIT,), jnp.int32),
            pltpu.VMEM((UNIT,), jnp.int32),
            pltpu.VMEM((UNIT,), jnp.int32),
            pltpu.VMEM((UNIT,), jnp.int32),
            pltpu.VMEM((CHUNK, DIM), jnp.float32),
            pltpu.VMEM((CHUNK, DIM), jnp.float32),
            pltpu.VMEM((CHUNK, DIM), jnp.float32),
            pltpu.VMEM((CHUNK, DIM), jnp.float32),
            pltpu.VMEM((DIM // 8, UC, 8, CHUNK), jnp.float32),
            pltpu.VMEM((DIM // 8, UC, 8, CHUNK), jnp.float32),
            pltpu.VMEM((DIM,), jnp.float32),
            pltpu.VMEM((DIM,), jnp.float32),
            pltpu.VMEM((DIM, LANES), jnp.float32),
            pltpu.VMEM((DIM, LANES), jnp.float32),
            pltpu.VMEM((DIM, LANES), jnp.float32),
            pltpu.SemaphoreType.DMA,
            pltpu.SemaphoreType.DMA,
            pltpu.SemaphoreType.DMA,
            pltpu.SemaphoreType.DMA,
            pltpu.SemaphoreType.DMA,
            pltpu.SemaphoreType.DMA,
            pltpu.SemaphoreType.DMA,
            pltpu.SemaphoreType.DMA,
            pltpu.SemaphoreType.DMA,
            pltpu.SemaphoreType.DMA,
        ],
    )
    return f(idx3, ts3, table, time_table, gamma, beta)


def kernel(indices, time_steps, table, time_table, gamma, beta):
    b, l = indices.shape
    n = b * l
    assert b % UNIT == 0
    nunit = (l * b // UNIT) // NW
    assert nunit * NW * UNIT == n and nunit % 2 == 0
    idx3 = jnp.transpose(indices).astype(jnp.int32).reshape(NW, nunit, UNIT)
    ts3 = jnp.transpose(time_steps).astype(jnp.int32).reshape(NW, nunit, UNIT)
    out5 = _run(idx3, ts3, table.astype(jnp.float32),
                time_table.astype(jnp.float32), gamma.astype(jnp.float32),
                beta.astype(jnp.float32), l, b, nunit)
    out = jnp.transpose(out5, (2, 4, 0, 1, 3)).reshape(b, l, DIM)
    return out


# speedup vs baseline: 1.4617x; 1.4445x over previous
import functools

import jax
import jax.numpy as jnp
from jax import lax
from jax.experimental import pallas as pl
from jax.experimental.pallas import tpu as pltpu
from jax.experimental.pallas import tpu_sc as plsc

DIM = 64
LANES = 16
NJ = DIM // LANES
NC = 2
NS = 16
NW = NC * NS
CHUNK = 128
UNROLL = 4


def _rsqrt16(x):
    i = plsc.bitcast(x, jnp.int32)
    i = jnp.int32(0x5F3759DF) - (i >> 1)
    y = plsc.bitcast(i, jnp.float32)
    half = x * jnp.float32(0.5)
    for _ in range(3):
        y = y * (jnp.float32(1.5) - half * y * y)
    return y


def _ln_chunk(e_ref, t_ref, o_ref, gvecs, bvecs):
    def body(it, carry):
        base = it * UNROLL
        for u in range(UNROLL):
            r = base + u
            h = [e_ref[r, pl.ds(16 * j, 16)] + t_ref[r, pl.ds(16 * j, 16)]
                 for j in range(NJ)]
            s = (h[0] + h[1]) + (h[2] + h[3])
            ss = (h[0] * h[0] + h[1] * h[1]) + (h[2] * h[2] + h[3] * h[3])
            tot = jnp.sum(s)
            sstot = jnp.sum(ss)
            mu = tot * jnp.float32(1.0 / DIM)
            var = sstot * jnp.float32(1.0 / DIM) - mu * mu
            xv = jnp.full((16,), var + jnp.float32(1e-5), jnp.float32)
            rstd = _rsqrt16(xv)
            muv = jnp.full((16,), mu, jnp.float32)
            for j in range(NJ):
                o_ref[r, pl.ds(16 * j, 16)] = (h[j] - muv) * rstd * gvecs[j] + bvecs[j]
        return carry

    lax.fori_loop(0, CHUNK // UNROLL, body, 0, unroll=False)


def _sc_body(idx_hbm, ts_hbm, table_hbm, ttable_hbm, gamma_hbm, beta_hbm,
             out_hbm, idx_v, ts_v, e0, e1, t0, t1, o0, o1, gam_v, bet_v,
             se0, se1, st0, st1, so0, so1, nchunk):
    wid = lax.axis_index("s") * NC + lax.axis_index("c")
    rows_per_w = nchunk * CHUNK
    wbase = wid * rows_per_w

    pltpu.sync_copy(idx_hbm.at[wid], idx_v)
    pltpu.sync_copy(ts_hbm.at[wid], ts_v)
    pltpu.sync_copy(gamma_hbm, gam_v)
    pltpu.sync_copy(beta_hbm, bet_v)
    gvecs = [gam_v[pl.ds(16 * j, 16)] for j in range(NJ)]
    bvecs = [bet_v[pl.ds(16 * j, 16)] for j in range(NJ)]

    ebufs, tbufs, obufs = (e0, e1), (t0, t1), (o0, o1)
    esems, tsems, osems = (se0, se1), (st0, st1), (so0, so1)

    def issue_gather(i, p):
        pltpu.async_copy(table_hbm.at[idx_v.at[i]], ebufs[p], esems[p])
        pltpu.async_copy(ttable_hbm.at[ts_v.at[i]], tbufs[p], tsems[p])

    def wait_gather(i, p):
        pltpu.make_async_copy(table_hbm.at[idx_v.at[i]], ebufs[p], esems[p]).wait()
        pltpu.make_async_copy(ttable_hbm.at[ts_v.at[i]], tbufs[p], tsems[p]).wait()

    def out_slice(i):
        return out_hbm.at[pl.ds(wbase + i * CHUNK, CHUNK)]

    def step(i, p, issue_next, wait_prev_scatter):
        wait_gather(i, p)
        if issue_next:
            issue_gather(i + 1, 1 - p)
        if wait_prev_scatter:
            pltpu.make_async_copy(obufs[p], out_slice(i - 2), osems[p]).wait()
        _ln_chunk(ebufs[p], tbufs[p], obufs[p], gvecs, bvecs)
        pltpu.async_copy(obufs[p], out_slice(i), osems[p])

    issue_gather(0, 0)
    step(0, 0, True, False)
    step(1, 1, True, False)

    def loop_body(g, carry):
        i = 2 * g
        step(i, 0, True, True)
        step(i + 1, 1, True, True)
        return carry

    lax.fori_loop(1, (nchunk - 2) // 2, loop_body, 0, unroll=False)

    step(nchunk - 2, 0, True, True)
    step(nchunk - 1, 1, False, True)
    pltpu.make_async_copy(obufs[0], out_slice(nchunk - 2), osems[0]).wait()
    pltpu.make_async_copy(obufs[1], out_slice(nchunk - 1), osems[1]).wait()


@functools.partial(jax.jit, static_argnames=("n", "nchunk"))
def _run(idx, ts, table, time_table, gamma, beta, n, nchunk):
    mesh = plsc.VectorSubcoreMesh(core_axis_name="c", subcore_axis_name="s",
                                  num_cores=NC, num_subcores=NS)
    body = functools.partial(_sc_body, nchunk=nchunk)
    f = pl.kernel(
        body,
        out_type=jax.ShapeDtypeStruct((n, DIM), jnp.float32),
        mesh=mesh,
        compiler_params=pltpu.CompilerParams(
            needs_layout_passes=False, use_tc_tiling_on_sc=False),
        scratch_types=[
            pltpu.VMEM((nchunk, CHUNK), jnp.int32),
            pltpu.VMEM((nchunk, CHUNK), jnp.int32),
            pltpu.VMEM((CHUNK, DIM), jnp.float32),
            pltpu.VMEM((CHUNK, DIM), jnp.float32),
            pltpu.VMEM((CHUNK, DIM), jnp.float32),
            pltpu.VMEM((CHUNK, DIM), jnp.float32),
            pltpu.VMEM((CHUNK, DIM), jnp.float32),
            pltpu.VMEM((CHUNK, DIM), jnp.float32),
            pltpu.VMEM((DIM,), jnp.float32),
            pltpu.VMEM((DIM,), jnp.float32),
            pltpu.SemaphoreType.DMA,
            pltpu.SemaphoreType.DMA,
            pltpu.SemaphoreType.DMA,
            pltpu.SemaphoreType.DMA,
            pltpu.SemaphoreType.DMA,
            pltpu.SemaphoreType.DMA,
        ],
    )
    return f(idx, ts, table, time_table, gamma, beta)


def kernel(indices, time_steps, table, time_table, gamma, beta):
    b, l = indices.shape
    n = b * l
    assert n % (NW * CHUNK) == 0
    nchunk = n // (NW * CHUNK)
    assert nchunk % 2 == 0
    idx = indices.reshape(NW, nchunk, CHUNK).astype(jnp.int32)
    ts = time_steps.reshape(NW, nchunk, CHUNK).astype(jnp.int32)
    out = _run(idx, ts, table.astype(jnp.float32), time_table.astype(jnp.float32),
               gamma.astype(jnp.float32), beta.astype(jnp.float32), n, nchunk)
    return out.reshape(b, l, DIM)
